# Initial kernel scaffold; baseline (speedup 1.0000x reference)
import jax
import jax.numpy as jnp
from jax.experimental import pallas as pl


def kernel(pro1_x, pro1_edge_index, pro1_batch, pro2_x, pro2_edge_index, pro2_batch, mas1_straight, mas1_flipped, mas2_straight, mas2_flipped, W_g1, b_g1, W_fc1, b_fc1, W_g2, b_g2, W_fc2, b_fc2, W_m1s, b_m1s, W_m1f, b_m1f, W_m2s, b_m2s, W_m2f, b_m2f, W_out, b_out):
    raise NotImplementedError("write your pallas kernel here")


# speedup vs baseline: 5.7048x; 5.7048x over previous
import functools

import jax
import jax.numpy as jnp
from jax import lax
from jax.experimental import pallas as pl
from jax.experimental.pallas import tpu as pltpu
from jax.experimental.pallas import tpu_sc as plsc

N_NODES = 10000
N_EDGES = 320000
NUM_GRAPHS = 32
D = 128
D_DESC = 80
L_DESC = 200

NP = 10112
CHUNK = 128
N_SUB = 16
N_CORES = 2
CHUNKS_PER_TILE = 160
E_TILE = CHUNKS_PER_TILE * CHUNK
E_PAD = E_TILE * N_SUB
ROWS_PER_TILE = NP // N_SUB

_sc_mesh = lambda: plsc.VectorSubcoreMesh(core_axis_name="c", subcore_axis_name="s")


def _deg_body(dst_hbm, zeros_hbm, ones_hbm, deg_out, dst_v, ones_v, acc_sh):
    c = lax.axis_index("c")
    s = lax.axis_index("s")
    r0 = s * ROWS_PER_TILE
    pltpu.sync_copy(zeros_hbm.at[pl.ds(r0, ROWS_PER_TILE)],
                    acc_sh.at[pl.ds(r0, ROWS_PER_TILE)])
    pltpu.sync_copy(ones_hbm, ones_v)
    pltpu.sync_copy(dst_hbm.at[c, s], dst_v)
    plsc.subcore_barrier()

    def body(j, carry):
        pltpu.sync_copy(ones_v, acc_sh.at[dst_v.at[j]], add=True)
        return carry

    lax.fori_loop(0, CHUNKS_PER_TILE, body, 0)
    plsc.subcore_barrier()

    def spin(i, carry):
        return carry * 3 + 1

    wait_val = lax.fori_loop(0, 100000, spin, 1)
    dst_v[0, pl.ds(0, 16)] = jnp.full((16,), wait_val, jnp.int32)
    plsc.subcore_barrier()
    pltpu.sync_copy(acc_sh.at[pl.ds(r0, ROWS_PER_TILE)],
                    deg_out.at[c, pl.ds(r0, ROWS_PER_TILE)])


def _deg_counts(dst_r, zeros16, ones16):
    kfn = functools.partial(
        pl.kernel,
        mesh=_sc_mesh(),
        out_type=jax.ShapeDtypeStruct((N_CORES, NP, 16), jnp.float32),
        scratch_types=[
            pltpu.VMEM((CHUNKS_PER_TILE, CHUNK), jnp.int32),
            pltpu.VMEM((CHUNK, 16), jnp.float32),
            pltpu.VMEM_SHARED((NP, 16), jnp.float32),
        ],
    )
    return kfn(_deg_body)(dst_r, zeros16, ones16)


def _spmm_body(src_hbm, dst_hbm, g_hbm, zeros_hbm, acc_out, dump_out,
               src_v, dst_v, rows_v, acc_sh, gsem):
    c = lax.axis_index("c")
    s = lax.axis_index("s")
    r0 = s * ROWS_PER_TILE
    pltpu.sync_copy(zeros_hbm.at[pl.ds(r0, ROWS_PER_TILE)],
                    acc_sh.at[pl.ds(r0, ROWS_PER_TILE)])
    pltpu.sync_copy(dst_hbm.at[c, s], dst_v)
    plsc.subcore_barrier()

    def gather_body(j, carry):
        pltpu.sync_copy(src_hbm.at[c, s, j], src_v)
        pltpu.async_copy(g_hbm.at[src_v], rows_v, gsem).wait()
        pltpu.sync_copy(
            rows_v, dump_out.at[c, pl.ds(s * E_TILE + j * CHUNK, CHUNK)])
        return carry

    lax.fori_loop(0, CHUNKS_PER_TILE, gather_body, 0)

    def scat_body(j, carry):
        pltpu.sync_copy(
            dump_out.at[c, pl.ds(s * E_TILE + j * CHUNK, CHUNK)], rows_v)
        pltpu.sync_copy(rows_v, acc_sh.at[dst_v.at[j]], add=True)
        return carry

    lax.fori_loop(0, CHUNKS_PER_TILE, scat_body, 0)
    plsc.subcore_barrier()

    def spin(i, carry):
        return carry * 3 + 1

    src_v[pl.ds(0, 16)] = jnp.full(
        (16,), lax.fori_loop(0, 100000, spin, 1), jnp.int32)
    plsc.subcore_barrier()
    pltpu.sync_copy(acc_sh.at[pl.ds(r0, ROWS_PER_TILE)],
                    acc_out.at[c, pl.ds(r0, ROWS_PER_TILE)])


def _spmm(src_r, dst_r, g_flat, zerosD):
    kfn = functools.partial(
        pl.kernel,
        mesh=_sc_mesh(),
        out_type=[jax.ShapeDtypeStruct((N_CORES, NP, D), jnp.float32),
                  jax.ShapeDtypeStruct((N_CORES, E_PAD, D), jnp.float32)],
        scratch_types=[
            pltpu.VMEM((CHUNK,), jnp.int32),
            pltpu.VMEM((CHUNKS_PER_TILE, CHUNK), jnp.int32),
            pltpu.VMEM((CHUNK, D), jnp.float32),
            pltpu.VMEM_SHARED((NP, D), jnp.float32),
            pltpu.SemaphoreType.DMA,
        ],
    )
    return kfn(_spmm_body)(src_r, dst_r, g_flat, zerosD)


def _h_body(x_ref, w_ref, o_ref):
    o_ref[0] = jnp.dot(x_ref[0], w_ref[0].T, preferred_element_type=jnp.float32)


def _h_matmul(xs, wgs):
    blk = 2000
    return pl.pallas_call(
        _h_body,
        grid=(2, N_NODES // blk),
        in_specs=[
            pl.BlockSpec((1, blk, D), lambda c, j: (c, j, 0)),
            pl.BlockSpec((1, D, D), lambda c, j: (c, 0, 0)),
        ],
        out_specs=pl.BlockSpec((1, blk, D), lambda c, j: (c, j, 0)),
        out_shape=jax.ShapeDtypeStruct((2, N_NODES, D), jnp.float32),
    )(xs, wgs)


def _g_body(h_ref, deg_ref, o_ref):
    dis = lax.rsqrt(deg_ref[0][:, 0:1] + 1.0)
    o_ref[0] = h_ref[0] * dis


def _g_scale(h, deg):
    blk = 2000
    return pl.pallas_call(
        _g_body,
        grid=(2, N_NODES // blk),
        in_specs=[
            pl.BlockSpec((1, blk, D), lambda c, j: (c, j, 0)),
            pl.BlockSpec((1, blk, 16), lambda c, j: (c, j, 0)),
        ],
        out_specs=pl.BlockSpec((1, blk, D), lambda c, j: (c, j, 0)),
        out_shape=jax.ShapeDtypeStruct((2, N_NODES, D), jnp.float32),
    )(h, deg)


def _lrelu(t):
    return jnp.where(t >= 0, t, 0.01 * t)


def _pool_body(acc_ref, h_ref, deg_ref, batch_ref, b_ref, sum_ref, cnt_ref):
    j = pl.program_id(1)
    deg = deg_ref[0][:, 0:1] + 1.0
    node = acc_ref[0] * lax.rsqrt(deg) + h_ref[0] / deg + b_ref[0, 0]
    node = _lrelu(node)
    bt = batch_ref[0, 0]
    onehot = (lax.broadcasted_iota(jnp.int32, (NUM_GRAPHS, bt.shape[0]), 0)
              == bt[None, :]).astype(jnp.float32)
    psum = jnp.dot(onehot, node, preferred_element_type=jnp.float32)
    pcnt = jnp.sum(onehot, axis=1)[:, None] * jnp.ones((1, D), jnp.float32)

    @pl.when(j == 0)
    def _():
        sum_ref[0] = jnp.zeros_like(sum_ref[0])
        cnt_ref[0] = jnp.zeros_like(cnt_ref[0])

    sum_ref[0] += psum
    cnt_ref[0] += pcnt


def _pool(acc, h, deg, batch_r, bgs):
    blk = 2000
    nj = N_NODES // blk
    return pl.pallas_call(
        _pool_body,
        grid=(2, nj),
        in_specs=[
            pl.BlockSpec((1, blk, D), lambda c, j: (c, j, 0)),
            pl.BlockSpec((1, blk, D), lambda c, j: (c, j, 0)),
            pl.BlockSpec((1, blk, 16), lambda c, j: (c, j, 0)),
            pl.BlockSpec((1, 1, blk), lambda c, j: (c * nj + j, 0, 0)),
            pl.BlockSpec((1, 1, D), lambda c, j: (c, 0, 0)),
        ],
        out_specs=[
            pl.BlockSpec((1, NUM_GRAPHS, D), lambda c, j: (c, 0, 0)),
            pl.BlockSpec((1, NUM_GRAPHS, D), lambda c, j: (c, 0, 0)),
        ],
        out_shape=[
            jax.ShapeDtypeStruct((2, NUM_GRAPHS, D), jnp.float32),
            jax.ShapeDtypeStruct((2, NUM_GRAPHS, D), jnp.float32),
        ],
    )(acc, h, deg, batch_r, bgs)


def _mas_body(m_ref, w_ref, b_ref, o_ref):
    l = pl.program_id(1)
    x = m_ref[0].reshape(NUM_GRAPHS * 40, D_DESC)
    y = jnp.dot(x, w_ref[0].T, preferred_element_type=jnp.float32) + b_ref[0, 0]
    y = _lrelu(y).reshape(NUM_GRAPHS, 40, D)
    ymax = jnp.max(y, axis=1)

    @pl.when(l == 0)
    def _():
        o_ref[0] = jnp.full_like(o_ref[0], -jnp.inf)

    o_ref[0] = jnp.maximum(o_ref[0], ymax)


def _mas(mas_all, wms, bms):
    return pl.pallas_call(
        _mas_body,
        grid=(4, L_DESC // 40),
        in_specs=[
            pl.BlockSpec((1, NUM_GRAPHS, 40, D_DESC), lambda b, l: (b, 0, l, 0)),
            pl.BlockSpec((1, D, D_DESC), lambda b, l: (b, 0, 0)),
            pl.BlockSpec((1, 1, D), lambda b, l: (b, 0, 0)),
        ],
        out_specs=pl.BlockSpec((1, NUM_GRAPHS, D), lambda b, l: (b, 0, 0)),
        out_shape=jax.ShapeDtypeStruct((4, NUM_GRAPHS, D), jnp.float32),
    )(mas_all, wms, bms)


def _final_body(sum_ref, cnt_ref, wfc_ref, bfc_ref, m_ref, wo_ref, bo_ref, o_ref):
    pooled = sum_ref[...] / jnp.maximum(cnt_ref[...], 1.0)
    z1 = _lrelu(jnp.dot(pooled[0], wfc_ref[0].T,
                        preferred_element_type=jnp.float32) + bfc_ref[0])
    z2 = _lrelu(jnp.dot(pooled[1], wfc_ref[1].T,
                        preferred_element_type=jnp.float32) + bfc_ref[1])
    combined = jnp.concatenate(
        [z1, z2, m_ref[0], m_ref[1], m_ref[2], m_ref[3]], axis=1)
    o_ref[...] = (jnp.sum(combined * wo_ref[...], axis=1, keepdims=True)
                  + bo_ref[...][None, :])


def _final(psum, pcnt, wfcs, bfcs, mas_out, W_out, b_out):
    return pl.pallas_call(
        _final_body,
        out_shape=jax.ShapeDtypeStruct((NUM_GRAPHS, 1), jnp.float32),
    )(psum, pcnt, wfcs, bfcs, mas_out, W_out, b_out)


def kernel(pro1_x, pro1_edge_index, pro1_batch, pro2_x, pro2_edge_index,
           pro2_batch, mas1_straight, mas1_flipped, mas2_straight,
           mas2_flipped, W_g1, b_g1, W_fc1, b_fc1, W_g2, b_g2, W_fc2, b_fc2,
           W_m1s, b_m1s, W_m1f, b_m1f, W_m2s, b_m2s, W_m2f, b_m2f,
           W_out, b_out):
    e1 = pro1_edge_index.astype(jnp.int32)
    e2 = pro2_edge_index.astype(jnp.int32)
    npad = E_PAD - N_EDGES
    pad_src = jnp.zeros((npad,), jnp.int32)
    pad_dst = N_NODES + (jnp.arange(npad, dtype=jnp.int32) % 16)
    src_r = jnp.stack([
        jnp.concatenate([e1[0], pad_src]),
        jnp.concatenate([e2[0], pad_src + N_NODES]),
    ]).reshape(N_CORES, N_SUB, CHUNKS_PER_TILE, CHUNK)
    dst_r = jnp.stack([
        jnp.concatenate([e1[1], pad_dst]),
        jnp.concatenate([e2[1], pad_dst]),
    ]).reshape(N_CORES, N_SUB, CHUNKS_PER_TILE, CHUNK)

    zeros16 = jnp.zeros((NP, 16), jnp.float32)
    ones16 = jnp.ones((CHUNK, 16), jnp.float32)
    zerosD = jnp.zeros((NP, D), jnp.float32)

    xs = jnp.stack([pro1_x, pro2_x])
    wgs = jnp.stack([W_g1, W_g2])
    bgs = jnp.stack([b_g1, b_g2])

    h = _h_matmul(xs, wgs)
    deg = _deg_counts(dst_r, zeros16, ones16)
    g = _g_scale(h, deg)
    acc, _ = _spmm(src_r, dst_r, g.reshape(2 * N_NODES, D), zerosD)

    batch_r = jnp.stack([pro1_batch.astype(jnp.int32),
                         pro2_batch.astype(jnp.int32)]).reshape(10, 1, 2000)
    psum, pcnt = _pool(acc[:, :N_NODES], h, deg[:, :N_NODES], batch_r,
                       bgs.reshape(2, 1, D))

    mas_all = jnp.stack([mas1_straight, mas1_flipped, mas2_straight,
                         mas2_flipped])
    wms = jnp.stack([W_m1s, W_m1f, W_m2s, W_m2f])
    bms = jnp.stack([b_m1s, b_m1f, b_m2s, b_m2f])
    mas_out = _mas(mas_all, wms, bms.reshape(4, 1, D))

    wfcs = jnp.stack([W_fc1, W_fc2])
    bfcs = jnp.stack([b_fc1, b_fc2])
    return _final(psum, pcnt, wfcs, bfcs, mas_out, W_out, b_out)
